# Initial kernel scaffold; baseline (speedup 1.0000x reference)
import jax
import jax.numpy as jnp
from jax.experimental import pallas as pl


def kernel(x, edge_index, edge_attr, v_lin0_W, v_lin0_b, e_lin0_W, e_lin0_b, v_lins1_W, v_lins1_b, v_lins2_W, v_lins2_b, v_lins3_W, v_lins3_b, v_lins4_W, v_lins4_b, e_lins0_W, e_lins0_b, v_bn_g, v_bn_b, e_bn_g, e_bn_b, par_W0, par_b0, par_W1, par_b1, par_W2, par_b2, z_W0, z_b0, z_W1, z_b1):
    raise NotImplementedError("write your pallas kernel here")


# speedup vs baseline: 2.6771x; 2.6771x over previous
import functools

import jax
import jax.numpy as jnp
from jax import lax
from jax.experimental import pallas as pl
from jax.experimental.pallas import tpu as pltpu
from jax.experimental.pallas import tpu_sc as plsc

N = 10000
E = 160000
U = 48
DEPTH = 12
K = 16

NP_ = 10240
SUB = NP_ // 16
EBLK = 4000
GRID = E // EBLK
CB = 256
NBLK = E // CB
ROWS = E // 128

F32 = jnp.float32


def _silu(t):
    return t * jax.nn.sigmoid(t)


def _pre_node_body(x_ref, w_ref, b_ref, h_ref):
    h_ref[...] = _silu(jnp.dot(x_ref[...], w_ref[...],
                               preferred_element_type=F32) + b_ref[...])


def _pre_edge_body(ea_ref, w_ref, b_ref, w0_ref, s_ref):
    t = _silu(ea_ref[...] * w_ref[...] + b_ref[...])
    w0_ref[...] = t
    s_ref[...] = jax.nn.sigmoid(t)


def _cnt_fin_body(cp_ref, cnt_ref):
    c = cp_ref[0:N, 0:1] + cp_ref[NP_:NP_ + N, 0:1]
    cnt_ref[...] = jnp.maximum(c, 1.0)


def _node_mm_body(h_ref, w1, b1, w2, b2, w3, b3, w4, b4, x1, x2, x3, x4):
    h = h_ref[...]
    x1[...] = jnp.dot(h, w1[...], preferred_element_type=F32) + b1[...]
    x2[...] = jnp.dot(h, w2[...], preferred_element_type=F32) + b2[...]
    x3[...] = jnp.dot(h, w3[...], preferred_element_type=F32) + b3[...]
    x4[...] = jnp.dot(h, w4[...], preferred_element_type=F32) + b4[...]


def _edge_stats_body(w_ref, y_ref, we_ref, be_ref, st_ref, s1, s2):
    i = pl.program_id(0)

    @pl.when(i == 0)
    def _():
        s1[...] = jnp.zeros_like(s1)
        s2[...] = jnp.zeros_like(s2)

    t = (jnp.dot(w_ref[...], we_ref[...], preferred_element_type=F32)
         + be_ref[...] + y_ref[...])
    s1[...] += jnp.sum(t, axis=0, keepdims=True)
    s2[...] += jnp.sum(t * t, axis=0, keepdims=True)

    @pl.when(i == GRID - 1)
    def _():
        st_ref[0:1, :] = s1[...]
        st_ref[1:2, :] = s2[...]


def _edge_update_body(w_ref, y_ref, we_ref, be_ref, st_ref, g_ref, b_ref,
                      wn_ref, sn_ref):
    t = (jnp.dot(w_ref[...], we_ref[...], preferred_element_type=F32)
         + be_ref[...] + y_ref[...])
    m = st_ref[0:1, :] * (1.0 / E)
    v = st_ref[1:2, :] * (1.0 / E) - m * m
    inv = lax.rsqrt(v + 1e-5)
    wn = w_ref[...] + _silu(g_ref[...] * (t - m) * inv + b_ref[...])
    wn_ref[...] = wn
    sn_ref[...] = jax.nn.sigmoid(wn)


def _node_update_body(x1_ref, ap_ref, cnt_ref, h_ref, g_ref, b_ref, hn_ref):
    agg = (ap_ref[0:N, :] + ap_ref[NP_:NP_ + N, :]) / cnt_ref[...]
    t = x1_ref[...] + agg
    m = jnp.mean(t, axis=0, keepdims=True)
    d = t - m
    v = jnp.mean(d * d, axis=0, keepdims=True)
    bn = g_ref[...] * d * lax.rsqrt(v + 1e-5) + b_ref[...]
    hn_ref[...] = h_ref[...] + _silu(bn)


def _post_body(w_ref, pw0, pb0, pw1, pb1, pw2, pb2, zw0, zb0, zw1, zb1,
               hh_ref, zs_ref, acc):
    i = pl.program_id(0)

    @pl.when(i == 0)
    def _():
        acc[...] = jnp.zeros_like(acc)

    w = w_ref[...]
    t = _silu(jnp.dot(w, pw0[...], preferred_element_type=F32) + pb0[...])
    t = _silu(jnp.dot(t, pw1[...], preferred_element_type=F32) + pb1[...])
    hh_ref[...] = jnp.dot(t, pw2[...], preferred_element_type=F32) + pb2[...]
    z = jnp.maximum(jnp.dot(w, zw0[...], preferred_element_type=F32)
                    + zb0[...], 0.0)
    z = jnp.dot(z, zw1[...], preferred_element_type=F32) + zb1[...]
    acc[...] += jnp.sum(z).reshape(1, 1)

    @pl.when(i == GRID - 1)
    def _():
        zs_ref[...] = acc[...]


def _softmax_body(hh_ref, zs_ref, heu_ref, lz_ref):
    a = hh_ref[...]
    m = jnp.max(a, axis=1, keepdims=True)
    e = jnp.exp(a - m)
    heu_ref[...] = e / jnp.sum(e, axis=1, keepdims=True)
    lz_ref[...] = zs_ref[...] * (1.0 / E)


_TC_PARAMS = pltpu.CompilerParams(dimension_semantics=("arbitrary",))


def _bfull(shape):
    return pl.BlockSpec(shape, lambda i: tuple(0 for _ in shape))


def _bedge(w):
    return pl.BlockSpec((EBLK, w), lambda i: (i, 0))


def _sc_cnt_body(srcr, cntp, acc, idxv, ones_v, zb, sem):
    cid = lax.axis_index("c")
    sid = lax.axis_index("s")
    w = cid * 16 + sid

    def fill_ones(i, _):
        ones_v[i, :] = jnp.ones((16,), F32)
        return 0
    lax.fori_loop(0, 128, fill_ones, 0)

    def fill_zero(i, _):
        zb[i, :] = jnp.zeros((16,), F32)
        return 0
    lax.fori_loop(0, SUB, fill_zero, 0)
    pltpu.sync_copy(zb, acc.at[pl.ds(sid * SUB, SUB)])
    plsc.subcore_barrier()

    nrow = jnp.where(w < 2, 40, 39)
    start = 39 * w + jnp.minimum(w, 2)

    def blk(j, _):
        r = start + j
        pltpu.sync_copy(srcr.at[pl.ds(r, 1)], idxv)
        pltpu.sync_copy(ones_v, acc.at[idxv.at[0]], add=True)
        return 0
    lax.fori_loop(0, nrow, blk, 0)
    plsc.subcore_barrier()
    pltpu.sync_copy(acc.at[pl.ds(sid * SUB, SUB)],
                    cntp.at[pl.ds(cid * NP_ + sid * SUB, SUB)])


@functools.cache
def _sc_cnt_kernel():
    mesh = plsc.VectorSubcoreMesh(core_axis_name="c", subcore_axis_name="s")
    return pl.kernel(
        _sc_cnt_body,
        out_type=jax.ShapeDtypeStruct((2 * NP_, 16), F32),
        mesh=mesh,
        scratch_types=[
            pltpu.VMEM_SHARED((NP_, 16), F32),
            pltpu.VMEM((1, 128), jnp.int32),
            pltpu.VMEM((128, 16), F32),
            pltpu.VMEM((SUB, 16), F32),
            pltpu.SemaphoreType.DMA,
        ],
        compiler_params=pltpu.CompilerParams(use_tc_tiling_on_sc=False),
    )


def _sc_cnt(srcr):
    return _sc_cnt_kernel()(srcr)


def _sc_layer_body(srcr, dstr, s_hbm, x2_hbm, x3_hbm, x4_hbm, y_hbm, aggp,
                   acc, srcv, dstv, g2, g3, g4, sv, zb, sem):
    cid = lax.axis_index("c")
    sid = lax.axis_index("s")
    w = cid * 16 + sid

    def fill_zero(i, _):
        for k in range(3):
            zb[i, pl.ds(k * 16, 16)] = jnp.zeros((16,), F32)
        return 0
    lax.fori_loop(0, SUB, fill_zero, 0)
    pltpu.sync_copy(zb, acc.at[pl.ds(sid * SUB, SUB)])
    plsc.subcore_barrier()

    nblk = jnp.where(w < 17, 20, 19)
    start = 19 * w + jnp.minimum(w, 17)

    def blk(j, _):
        b = start + j
        ebase = b * CB
        row = b * 2
        pltpu.sync_copy(srcr.at[pl.ds(row, 2)], srcv)
        pltpu.sync_copy(dstr.at[pl.ds(row, 2)], dstv)
        cps = [
            pltpu.async_copy(x2_hbm.at[dstv.at[0]], g2.at[pl.ds(0, 128)], sem),
            pltpu.async_copy(x2_hbm.at[dstv.at[1]], g2.at[pl.ds(128, 128)], sem),
            pltpu.async_copy(x3_hbm.at[srcv.at[0]], g3.at[pl.ds(0, 128)], sem),
            pltpu.async_copy(x3_hbm.at[srcv.at[1]], g3.at[pl.ds(128, 128)], sem),
            pltpu.async_copy(x4_hbm.at[dstv.at[0]], g4.at[pl.ds(0, 128)], sem),
            pltpu.async_copy(x4_hbm.at[dstv.at[1]], g4.at[pl.ds(128, 128)], sem),
            pltpu.async_copy(s_hbm.at[pl.ds(ebase, CB)], sv, sem),
        ]
        for cp in cps:
            cp.wait()

        def comp(i, _):
            for k in range(3):
                sl = pl.ds(k * 16, 16)
                g2[i, sl] = sv[i, sl] * g2[i, sl]
                g3[i, sl] = g3[i, sl] + g4[i, sl]
            return 0
        lax.fori_loop(0, CB, comp, 0)

        pltpu.sync_copy(g3, y_hbm.at[pl.ds(ebase, CB)])
        pltpu.sync_copy(g2.at[pl.ds(0, 128)], acc.at[srcv.at[0]], add=True)
        pltpu.sync_copy(g2.at[pl.ds(128, 128)], acc.at[srcv.at[1]], add=True)
        return 0
    lax.fori_loop(0, nblk, blk, 0)
    plsc.subcore_barrier()
    pltpu.sync_copy(acc.at[pl.ds(sid * SUB, SUB)],
                    aggp.at[pl.ds(cid * NP_ + sid * SUB, SUB)])


@functools.cache
def _sc_layer_kernel():
    mesh = plsc.VectorSubcoreMesh(core_axis_name="c", subcore_axis_name="s")
    return pl.kernel(
        _sc_layer_body,
        out_type=(jax.ShapeDtypeStruct((E, U), F32),
                  jax.ShapeDtypeStruct((2 * NP_, U), F32)),
        mesh=mesh,
        scratch_types=[
            pltpu.VMEM_SHARED((NP_, U), F32),
            pltpu.VMEM((2, 128), jnp.int32),
            pltpu.VMEM((2, 128), jnp.int32),
            pltpu.VMEM((CB, U), F32),
            pltpu.VMEM((CB, U), F32),
            pltpu.VMEM((CB, U), F32),
            pltpu.VMEM((CB, U), F32),
            pltpu.VMEM((SUB, U), F32),
            pltpu.SemaphoreType.DMA,
        ],
        compiler_params=pltpu.CompilerParams(use_tc_tiling_on_sc=False),
    )


def _sc_layer(srcr, dstr, s, x2, x3, x4):
    return _sc_layer_kernel()(srcr, dstr, s, x2, x3, x4)


def kernel(x, edge_index, edge_attr, v_lin0_W, v_lin0_b, e_lin0_W, e_lin0_b,
           v_lins1_W, v_lins1_b, v_lins2_W, v_lins2_b, v_lins3_W, v_lins3_b,
           v_lins4_W, v_lins4_b, e_lins0_W, e_lins0_b, v_bn_g, v_bn_b,
           e_bn_g, e_bn_b, par_W0, par_b0, par_W1, par_b1, par_W2, par_b2,
           z_W0, z_b0, z_W1, z_b1):
    src_r = edge_index[0].reshape(ROWS, 128)
    dst_r = edge_index[1].reshape(ROWS, 128)
    r2 = lambda b: b.reshape(1, -1)

    h = pl.pallas_call(
        _pre_node_body,
        out_shape=jax.ShapeDtypeStruct((N, U), F32),
    )(x, v_lin0_W, r2(v_lin0_b))

    w, s = pl.pallas_call(
        _pre_edge_body,
        grid=(GRID,),
        in_specs=[_bedge(1), _bfull((1, U)), _bfull((1, U))],
        out_specs=[_bedge(U), _bedge(U)],
        out_shape=[jax.ShapeDtypeStruct((E, U), F32),
                   jax.ShapeDtypeStruct((E, U), F32)],
        compiler_params=_TC_PARAMS,
    )(edge_attr, e_lin0_W, r2(e_lin0_b))

    cntp = _sc_cnt(src_r)
    cnt = pl.pallas_call(
        _cnt_fin_body,
        out_shape=jax.ShapeDtypeStruct((N, 1), F32),
    )(cntp)

    for i in range(DEPTH):
        x1, x2, x3, x4 = pl.pallas_call(
            _node_mm_body,
            out_shape=[jax.ShapeDtypeStruct((N, U), F32)] * 4,
        )(h, v_lins1_W[i], r2(v_lins1_b[i]), v_lins2_W[i], r2(v_lins2_b[i]),
          v_lins3_W[i], r2(v_lins3_b[i]), v_lins4_W[i], r2(v_lins4_b[i]))

        y, aggp = _sc_layer(src_r, dst_r, s, x2, x3, x4)

        stats = pl.pallas_call(
            _edge_stats_body,
            grid=(GRID,),
            in_specs=[_bedge(U), _bedge(U), _bfull((U, U)), _bfull((1, U))],
            out_specs=_bfull((2, U)),
            out_shape=jax.ShapeDtypeStruct((2, U), F32),
            scratch_shapes=[pltpu.VMEM((1, U), F32), pltpu.VMEM((1, U), F32)],
            compiler_params=_TC_PARAMS,
        )(w, y, e_lins0_W[i], r2(e_lins0_b[i]))

        w, s = pl.pallas_call(
            _edge_update_body,
            grid=(GRID,),
            in_specs=[_bedge(U), _bedge(U), _bfull((U, U)), _bfull((1, U)),
                      _bfull((2, U)), _bfull((1, U)), _bfull((1, U))],
            out_specs=[_bedge(U), _bedge(U)],
            out_shape=[jax.ShapeDtypeStruct((E, U), F32),
                       jax.ShapeDtypeStruct((E, U), F32)],
            compiler_params=_TC_PARAMS,
        )(w, y, e_lins0_W[i], r2(e_lins0_b[i]), stats,
          r2(e_bn_g[i]), r2(e_bn_b[i]))

        h = pl.pallas_call(
            _node_update_body,
            out_shape=jax.ShapeDtypeStruct((N, U), F32),
        )(x1, aggp, cnt, h, r2(v_bn_g[i]), r2(v_bn_b[i]))

    hh, zsum = pl.pallas_call(
        _post_body,
        grid=(GRID,),
        in_specs=[_bedge(U)] + [_bfull(t) for t in
                  [(U, U), (1, U), (U, U), (1, U), (U, 1), (1, 1),
                   (U, U), (1, U), (U, 1), (1, 1)]],
        out_specs=[_bedge(1), _bfull((1, 1))],
        out_shape=[jax.ShapeDtypeStruct((E, 1), F32),
                   jax.ShapeDtypeStruct((1, 1), F32)],
        scratch_shapes=[pltpu.VMEM((1, 1), F32)],
        compiler_params=_TC_PARAMS,
    )(w, par_W0, r2(par_b0), par_W1, r2(par_b1), par_W2,
      par_b2.reshape(1, 1), z_W0, r2(z_b0), z_W1, z_b1.reshape(1, 1))

    heu, logz = pl.pallas_call(
        _softmax_body,
        out_shape=[jax.ShapeDtypeStruct((E // K, K), F32),
                   jax.ShapeDtypeStruct((1, 1), F32)],
    )(hh.reshape(E // K, K), zsum)

    return heu.reshape(E), logz.reshape(1)
